# baseline (speedup 1.0000x reference)
import numpy as np
import jax
import jax.numpy as jnp
from jax.experimental import pallas as pl

_NSP = 4
_RCR = 5.2
_RCA = 3.5
_ETAR = 16.0
_ETAA = 8.0
_SHFR = (0.9 + 0.26875 * np.arange(16)).astype(np.float32)
_SHFA = np.array([0.9, 1.55, 2.2, 2.85], dtype=np.float32)
_SHFZ = ((np.arange(8) + 0.5) * np.pi / 8.0).astype(np.float32)
_COSZ = np.cos(_SHFZ)
_SINZ = np.sin(_SHFZ)
_K = 24
_NQ = 10

_JJ, _KK = np.triu_indices(_K, 1)
_P = _JJ.size
_PW = 384
_NSEG = 7
_SELJ6 = np.zeros((_NSEG * _K, _NSEG * _PW), dtype=np.float32)
_SELK6 = np.zeros((_NSEG * _K, _NSEG * _PW), dtype=np.float32)
for _g in range(_NSEG):
    _SELJ6[_g * _K + _JJ, _g * _PW + np.arange(_P)] = 1.0
    _SELK6[_g * _K + _KK, _g * _PW + np.arange(_P)] = 1.0

_BI = 128


def _cut(d, rc):
    return jnp.where(d < rc, 0.5 * jnp.cos(np.float32(np.pi / rc) * d) + 0.5, 0.0)


def _pow32(x):
    x = x * x
    x = x * x
    x = x * x
    x = x * x
    return x * x


def _ani_kernel(posi_ref, posT_ref, spcol_ref, pm_ref, selj_ref, selk_ref, out_ref):
    i = pl.program_id(0)
    bi = posi_ref.shape[0]
    n = posT_ref.shape[1]

    xi = posi_ref[:, 0:1]
    yi = posi_ref[:, 1:2]
    zi = posi_ref[:, 2:3]
    xr = posT_ref[0:1, :]
    yr = posT_ref[1:2, :]
    zr = posT_ref[2:3, :]

    dxm = xi - xr
    dym = yi - yr
    dzm = zi - zr
    d2m = dxm * dxm + dym * dym + dzm * dzm + 1e-12

    jio_i = jax.lax.broadcasted_iota(jnp.int32, (bi, n), 1)
    row_i = jax.lax.broadcasted_iota(jnp.int32, (bi, n), 0) + i * bi
    d2m = jnp.where(jio_i == row_i, 1e12, d2m)
    dist = jnp.sqrt(d2m)

    fc4 = 0.25 * _cut(dist, _RCR)
    oh = (spcol_ref[:, 0:1] == jax.lax.broadcasted_iota(jnp.int32, (n, _NSP), 1)).astype(jnp.float32)
    rads = []
    for m in range(16):
        dd = dist - _SHFR[m]
        t = jnp.exp(-_ETAR * dd * dd) * fc4
        rads.append(jax.lax.dot_general(t, oh, (((1,), (0,)), ((), ())),
                                        preferred_element_type=jnp.float32))
    radial = jnp.stack(rads, axis=2).reshape(bi, _NSP * 16)

    dwork = jnp.where(dist < _RCA, dist, 1e6)
    pm = pm_ref[...]
    pmhi = pm.astype(jnp.bfloat16)
    pmlo = (pm - pmhi.astype(jnp.float32)).astype(jnp.bfloat16)
    pmcat = jnp.concatenate([pmhi, pmlo], axis=1)
    nds, nvals = [], []
    for _ in range(_K):
        mval = jnp.min(dwork, axis=1, keepdims=True)
        aidx = jnp.argmin(dwork, axis=1, keepdims=True)
        sel = (jio_i == aidx)
        nds.append(mval)
        v8 = jax.lax.dot_general(sel.astype(jnp.bfloat16), pmcat,
                                 (((1,), (0,)), ((), ())),
                                 preferred_element_type=jnp.float32)
        nvals.append(v8[:, 0:4] + v8[:, 4:8])
        dwork = jnp.where(sel, 1e6, dwork)
    nd = jnp.concatenate(nds, axis=1)
    nx = jnp.concatenate([v[:, 0:1] for v in nvals], axis=1) - xi
    ny = jnp.concatenate([v[:, 1:2] for v in nvals], axis=1) - yi
    nz = jnp.concatenate([v[:, 2:3] for v in nvals], axis=1) - zi
    ns = jnp.concatenate([v[:, 3:4] for v in nvals], axis=1)
    vf = (nd < _RCA).astype(jnp.float32)
    fcn = _cut(nd, _RCA)

    big = jnp.concatenate([nd, nx, ny, nz, ns, vf, fcn], axis=1)
    bj = jax.lax.dot_general(big, selj_ref[...], (((1,), (0,)), ((), ())),
                             preferred_element_type=jnp.float32, precision=jax.lax.Precision.HIGHEST)
    bk = jax.lax.dot_general(big, selk_ref[...], (((1,), (0,)), ((), ())),
                             preferred_element_type=jnp.float32, precision=jax.lax.Precision.HIGHEST)
    d1 = bj[:, 0 * _PW:1 * _PW]
    x1 = bj[:, 1 * _PW:2 * _PW]
    y1 = bj[:, 2 * _PW:3 * _PW]
    z1 = bj[:, 3 * _PW:4 * _PW]
    s1 = bj[:, 4 * _PW:5 * _PW]
    v1 = bj[:, 5 * _PW:6 * _PW]
    d2 = bk[:, 0 * _PW:1 * _PW]
    x2 = bk[:, 1 * _PW:2 * _PW]
    y2 = bk[:, 2 * _PW:3 * _PW]
    z2 = bk[:, 3 * _PW:4 * _PW]
    s2 = bk[:, 4 * _PW:5 * _PW]
    v2 = bk[:, 5 * _PW:6 * _PW]
    fc1 = bj[:, 6 * _PW:7 * _PW]
    fc2 = bk[:, 6 * _PW:7 * _PW]

    dotp = x1 * x2 + y1 * y2 + z1 * z2
    c = 0.95 * dotp / (d1 * d2 + 1e-12)
    c = jnp.clip(c, -0.999999, 0.999999)
    s = jnp.sqrt(1.0 - c * c)
    base = 2.0 * fc1 * fc2 * (v1 * v2)

    davg = 0.5 * (d1 + d2)
    f2 = []
    for q in range(4):
        dd = davg - _SHFA[q]
        f2.append(jnp.exp(-_ETAA * dd * dd))
    ch = 0.5 * c
    sh = 0.5 * s
    f1 = []
    for z in range(8):
        u = jnp.maximum(0.5 + ch * _COSZ[z] + sh * _SINZ[z], 0.0)
        f1.append(jnp.exp2(32.0 * jnp.log2(u)))

    a = jnp.minimum(s1, s2)
    b = jnp.maximum(s1, s2)
    pidx = a * 4.0 - a * (a - 1.0) * 0.5 + (b - a)

    g = [[f1[z] * f2[t] for t in range(4)] for z in range(8)]
    cols = []
    for q in range(_NQ):
        wq = jnp.where(pidx == np.float32(q), base, 0.0)
        for z in range(8):
            for t in range(4):
                cols.append(jnp.sum(wq * g[z][t], axis=1, keepdims=True))
    angular = jnp.concatenate(cols, axis=1)

    out_ref[0] = jnp.concatenate([radial, angular], axis=1)


def kernel(species, positions):
    sp = species[0]
    pos = positions[0]
    n = pos.shape[0]
    posT = pos.T
    spcol = sp[:, None]
    pm = jnp.concatenate([pos, sp[:, None].astype(jnp.float32)], axis=1)
    selj = jnp.asarray(_SELJ6)
    selk = jnp.asarray(_SELK6)

    feats = pl.pallas_call(
        _ani_kernel,
        grid=(n // _BI,),
        in_specs=[
            pl.BlockSpec((_BI, 3), lambda i: (i, 0)),
            pl.BlockSpec((3, n), lambda i: (0, 0)),
            pl.BlockSpec((n, 1), lambda i: (0, 0)),
            pl.BlockSpec((n, 4), lambda i: (0, 0)),
            pl.BlockSpec((_NSEG * _K, _NSEG * _PW), lambda i: (0, 0)),
            pl.BlockSpec((_NSEG * _K, _NSEG * _PW), lambda i: (0, 0)),
        ],
        out_specs=pl.BlockSpec((1, _BI, 384), lambda i: (0, i, 0)),
        out_shape=jax.ShapeDtypeStruct((1, n, 384), jnp.float32),
    )(pos, posT, spcol, pm, selj, selk)

    return species, feats


# speedup vs baseline: 1.6340x; 1.6340x over previous
import numpy as np
import jax
import jax.numpy as jnp
from jax.experimental import pallas as pl

_NSP = 4
_RCR = 5.2
_RCA = 3.5
_ETAR = 16.0
_ETAA = 8.0
_SHFR = (0.9 + 0.26875 * np.arange(16)).astype(np.float32)
_SHFA = np.array([0.9, 1.55, 2.2, 2.85], dtype=np.float32)
_SHFZ = ((np.arange(8) + 0.5) * np.pi / 8.0).astype(np.float32)
_COSZ = np.cos(_SHFZ)
_SINZ = np.sin(_SHFZ)
_K = 24
_NQ = 10

_JJ, _KK = np.triu_indices(_K, 1)
_P = _JJ.size
_PW = 384
_NSEG = 7
_SELJ6 = np.zeros((_NSEG * _K, _NSEG * _PW), dtype=np.float32)
_SELK6 = np.zeros((_NSEG * _K, _NSEG * _PW), dtype=np.float32)
for _g in range(_NSEG):
    _SELJ6[_g * _K + _JJ, _g * _PW + np.arange(_P)] = 1.0
    _SELK6[_g * _K + _KK, _g * _PW + np.arange(_P)] = 1.0

_BI = 128


def _cut(d, rc):
    return jnp.where(d < rc, 0.5 * jnp.cos(np.float32(np.pi / rc) * d) + 0.5, 0.0)


def _pow32(x):
    x = x * x
    x = x * x
    x = x * x
    x = x * x
    return x * x


def _ani_kernel(posi_ref, posT_ref, spcol_ref, pm_ref, selj_ref, selk_ref, out_ref):
    i = pl.program_id(0)
    bi = posi_ref.shape[0]
    n = posT_ref.shape[1]

    xi = posi_ref[:, 0:1]
    yi = posi_ref[:, 1:2]
    zi = posi_ref[:, 2:3]
    xr = posT_ref[0:1, :]
    yr = posT_ref[1:2, :]
    zr = posT_ref[2:3, :]

    dxm = xi - xr
    dym = yi - yr
    dzm = zi - zr
    d2m = dxm * dxm + dym * dym + dzm * dzm + 1e-12

    jio_i = jax.lax.broadcasted_iota(jnp.int32, (bi, n), 1)
    row_i = jax.lax.broadcasted_iota(jnp.int32, (bi, n), 0) + i * bi
    d2m = jnp.where(jio_i == row_i, 1e12, d2m)
    dist = jnp.sqrt(d2m)

    fc4 = 0.25 * _cut(dist, _RCR)
    oh = (spcol_ref[:, 0:1] == jax.lax.broadcasted_iota(jnp.int32, (n, _NSP), 1)).astype(jnp.float32)
    rads = []
    for m in range(16):
        dd = dist - _SHFR[m]
        t = jnp.exp(-_ETAR * dd * dd) * fc4
        rads.append(jax.lax.dot_general(t, oh, (((1,), (0,)), ((), ())),
                                        preferred_element_type=jnp.float32))
    radial = jnp.stack(rads, axis=2).reshape(bi, _NSP * 16)

    jio_f = jio_i.astype(jnp.float32)
    dwork = jnp.where(dist < _RCA, dist, 1e6)
    pm = pm_ref[...]
    pmhi = pm.astype(jnp.bfloat16)
    pmlo = (pm - pmhi.astype(jnp.float32)).astype(jnp.bfloat16)
    pmcat = jnp.concatenate([pmhi, pmlo], axis=1)
    nds, nvals = [], []
    for _ in range(_K):
        mval = jnp.min(dwork, axis=1, keepdims=True)
        aidx = jnp.min(jnp.where(dwork == mval, jio_f, 1e9), axis=1, keepdims=True)
        sel = (jio_f == aidx)
        nds.append(mval)
        v8 = jax.lax.dot_general(sel.astype(jnp.bfloat16), pmcat,
                                 (((1,), (0,)), ((), ())),
                                 preferred_element_type=jnp.float32)
        nvals.append(v8[:, 0:4] + v8[:, 4:8])
        dwork = jnp.where(sel, 1e6, dwork)
    nd = jnp.concatenate(nds, axis=1)
    nx = jnp.concatenate([v[:, 0:1] for v in nvals], axis=1) - xi
    ny = jnp.concatenate([v[:, 1:2] for v in nvals], axis=1) - yi
    nz = jnp.concatenate([v[:, 2:3] for v in nvals], axis=1) - zi
    ns = jnp.concatenate([v[:, 3:4] for v in nvals], axis=1)
    vf = (nd < _RCA).astype(jnp.float32)
    fcn = _cut(nd, _RCA)

    big = jnp.concatenate([nd, nx, ny, nz, ns, vf, fcn], axis=1)
    bj = jax.lax.dot_general(big, selj_ref[...], (((1,), (0,)), ((), ())),
                             preferred_element_type=jnp.float32, precision=jax.lax.Precision.HIGHEST)
    bk = jax.lax.dot_general(big, selk_ref[...], (((1,), (0,)), ((), ())),
                             preferred_element_type=jnp.float32, precision=jax.lax.Precision.HIGHEST)
    d1 = bj[:, 0 * _PW:1 * _PW]
    x1 = bj[:, 1 * _PW:2 * _PW]
    y1 = bj[:, 2 * _PW:3 * _PW]
    z1 = bj[:, 3 * _PW:4 * _PW]
    s1 = bj[:, 4 * _PW:5 * _PW]
    v1 = bj[:, 5 * _PW:6 * _PW]
    d2 = bk[:, 0 * _PW:1 * _PW]
    x2 = bk[:, 1 * _PW:2 * _PW]
    y2 = bk[:, 2 * _PW:3 * _PW]
    z2 = bk[:, 3 * _PW:4 * _PW]
    s2 = bk[:, 4 * _PW:5 * _PW]
    v2 = bk[:, 5 * _PW:6 * _PW]
    fc1 = bj[:, 6 * _PW:7 * _PW]
    fc2 = bk[:, 6 * _PW:7 * _PW]

    dotp = x1 * x2 + y1 * y2 + z1 * z2
    c = 0.95 * dotp / (d1 * d2 + 1e-12)
    c = jnp.clip(c, -0.999999, 0.999999)
    s = jnp.sqrt(1.0 - c * c)
    base = 2.0 * fc1 * fc2 * (v1 * v2)

    davg = 0.5 * (d1 + d2)
    f2 = []
    for q in range(4):
        dd = davg - _SHFA[q]
        f2.append(jnp.exp(-_ETAA * dd * dd))
    ch = 0.5 * c
    sh = 0.5 * s
    f1 = []
    for z in range(8):
        u = jnp.maximum(0.5 + ch * _COSZ[z] + sh * _SINZ[z], 0.0)
        f1.append(jnp.exp2(32.0 * jnp.log2(u)))

    a = jnp.minimum(s1, s2)
    b = jnp.maximum(s1, s2)
    pidx = a * 4.0 - a * (a - 1.0) * 0.5 + (b - a)

    g = [[f1[z] * f2[t] for t in range(4)] for z in range(8)]
    cols = []
    for q in range(_NQ):
        wq = jnp.where(pidx == np.float32(q), base, 0.0)
        for z in range(8):
            for t in range(4):
                cols.append(jnp.sum(wq * g[z][t], axis=1, keepdims=True))
    angular = jnp.concatenate(cols, axis=1)

    out_ref[0] = jnp.concatenate([radial, angular], axis=1)


def kernel(species, positions):
    sp = species[0]
    pos = positions[0]
    n = pos.shape[0]
    posT = pos.T
    spcol = sp[:, None]
    pm = jnp.concatenate([pos, sp[:, None].astype(jnp.float32)], axis=1)
    selj = jnp.asarray(_SELJ6)
    selk = jnp.asarray(_SELK6)

    feats = pl.pallas_call(
        _ani_kernel,
        grid=(n // _BI,),
        in_specs=[
            pl.BlockSpec((_BI, 3), lambda i: (i, 0)),
            pl.BlockSpec((3, n), lambda i: (0, 0)),
            pl.BlockSpec((n, 1), lambda i: (0, 0)),
            pl.BlockSpec((n, 4), lambda i: (0, 0)),
            pl.BlockSpec((_NSEG * _K, _NSEG * _PW), lambda i: (0, 0)),
            pl.BlockSpec((_NSEG * _K, _NSEG * _PW), lambda i: (0, 0)),
        ],
        out_specs=pl.BlockSpec((1, _BI, 384), lambda i: (0, i, 0)),
        out_shape=jax.ShapeDtypeStruct((1, n, 384), jnp.float32),
    )(pos, posT, spcol, pm, selj, selk)

    return species, feats


# speedup vs baseline: 1.7400x; 1.0649x over previous
import numpy as np
import jax
import jax.numpy as jnp
from jax.experimental import pallas as pl

_NSP = 4
_RCR = 5.2
_RCA = 3.5
_ETAR = 16.0
_ETAA = 8.0
_SHFR = (0.9 + 0.26875 * np.arange(16)).astype(np.float32)
_SHFA = np.array([0.9, 1.55, 2.2, 2.85], dtype=np.float32)
_SHFZ = ((np.arange(8) + 0.5) * np.pi / 8.0).astype(np.float32)
_COSZ = np.cos(_SHFZ)
_SINZ = np.sin(_SHFZ)
_K = 24
_NQ = 10

_JJ, _KK = np.triu_indices(_K, 1)
_P = _JJ.size
_PW = 384
_NSEG = 7
_SELJ6 = np.zeros((_NSEG * _K, _NSEG * _PW), dtype=np.float32)
_SELK6 = np.zeros((_NSEG * _K, _NSEG * _PW), dtype=np.float32)
for _g in range(_NSEG):
    _SELJ6[_g * _K + _JJ, _g * _PW + np.arange(_P)] = 1.0
    _SELK6[_g * _K + _KK, _g * _PW + np.arange(_P)] = 1.0

_BI = 128


def _cut(d, rc):
    return jnp.where(d < rc, 0.5 * jnp.cos(np.float32(np.pi / rc) * d) + 0.5, 0.0)


def _pow32(x):
    x = x * x
    x = x * x
    x = x * x
    x = x * x
    return x * x


def _ani_kernel(posi_ref, posT_ref, spcol_ref, pm_ref, selj_ref, selk_ref, out_ref):
    i = pl.program_id(0)
    bi = posi_ref.shape[0]
    n = posT_ref.shape[1]

    xi = posi_ref[:, 0:1]
    yi = posi_ref[:, 1:2]
    zi = posi_ref[:, 2:3]
    xr = posT_ref[0:1, :]
    yr = posT_ref[1:2, :]
    zr = posT_ref[2:3, :]

    dxm = xi - xr
    dym = yi - yr
    dzm = zi - zr
    d2m = dxm * dxm + dym * dym + dzm * dzm + 1e-12

    jio_i = jax.lax.broadcasted_iota(jnp.int32, (bi, n), 1)
    row_i = jax.lax.broadcasted_iota(jnp.int32, (bi, n), 0) + i * bi
    d2m = jnp.where(jio_i == row_i, 1e12, d2m)
    dist = jnp.sqrt(d2m)

    fc4 = 0.25 * _cut(dist, _RCR)
    oh = (spcol_ref[:, 0:1] == jax.lax.broadcasted_iota(jnp.int32, (n, _NSP), 1)).astype(jnp.float32)
    rads = []
    for m in range(16):
        dd = dist - _SHFR[m]
        t = jnp.exp(-_ETAR * dd * dd) * fc4
        rads.append(jax.lax.dot_general(t, oh, (((1,), (0,)), ((), ())),
                                        preferred_element_type=jnp.float32))
    radial = jnp.stack(rads, axis=2).reshape(bi, _NSP * 16)

    jio_f = jio_i.astype(jnp.float32)
    dwork = jnp.where(dist < _RCA, dist, 1e6)
    spf_row = pm_ref[0:1, :]
    nds, ndxl, ndyl, ndzl, nspl = [], [], [], [], []
    for _ in range(_K):
        mval = jnp.min(dwork, axis=1, keepdims=True)
        aidx = jnp.min(jnp.where(dwork == mval, jio_f, 1e9), axis=1, keepdims=True)
        sel = (jio_f == aidx)
        nds.append(mval)
        ndxl.append(jnp.sum(jnp.where(sel, dxm, 0.0), axis=1, keepdims=True))
        ndyl.append(jnp.sum(jnp.where(sel, dym, 0.0), axis=1, keepdims=True))
        ndzl.append(jnp.sum(jnp.where(sel, dzm, 0.0), axis=1, keepdims=True))
        nspl.append(jnp.sum(jnp.where(sel, spf_row, 0.0), axis=1, keepdims=True))
        dwork = jnp.where(sel, 1e6, dwork)
    nd = jnp.concatenate(nds, axis=1)
    nx = -jnp.concatenate(ndxl, axis=1)
    ny = -jnp.concatenate(ndyl, axis=1)
    nz = -jnp.concatenate(ndzl, axis=1)
    ns = jnp.concatenate(nspl, axis=1)
    vf = (nd < _RCA).astype(jnp.float32)
    fcn = _cut(nd, _RCA)

    big = jnp.concatenate([nd, nx, ny, nz, ns, vf, fcn], axis=1)
    bj = jax.lax.dot_general(big, selj_ref[...], (((1,), (0,)), ((), ())),
                             preferred_element_type=jnp.float32, precision=jax.lax.Precision.HIGHEST)
    bk = jax.lax.dot_general(big, selk_ref[...], (((1,), (0,)), ((), ())),
                             preferred_element_type=jnp.float32, precision=jax.lax.Precision.HIGHEST)
    d1 = bj[:, 0 * _PW:1 * _PW]
    x1 = bj[:, 1 * _PW:2 * _PW]
    y1 = bj[:, 2 * _PW:3 * _PW]
    z1 = bj[:, 3 * _PW:4 * _PW]
    s1 = bj[:, 4 * _PW:5 * _PW]
    v1 = bj[:, 5 * _PW:6 * _PW]
    d2 = bk[:, 0 * _PW:1 * _PW]
    x2 = bk[:, 1 * _PW:2 * _PW]
    y2 = bk[:, 2 * _PW:3 * _PW]
    z2 = bk[:, 3 * _PW:4 * _PW]
    s2 = bk[:, 4 * _PW:5 * _PW]
    v2 = bk[:, 5 * _PW:6 * _PW]
    fc1 = bj[:, 6 * _PW:7 * _PW]
    fc2 = bk[:, 6 * _PW:7 * _PW]

    dotp = x1 * x2 + y1 * y2 + z1 * z2
    c = 0.95 * dotp / (d1 * d2 + 1e-12)
    c = jnp.clip(c, -0.999999, 0.999999)
    s = jnp.sqrt(1.0 - c * c)
    base = 2.0 * fc1 * fc2 * (v1 * v2)

    davg = 0.5 * (d1 + d2)
    f2 = []
    for q in range(4):
        dd = davg - _SHFA[q]
        f2.append(jnp.exp(-_ETAA * dd * dd))
    ch = 0.5 * c
    sh = 0.5 * s
    f1 = []
    for z in range(8):
        u = jnp.maximum(0.5 + ch * _COSZ[z] + sh * _SINZ[z], 0.0)
        f1.append(jnp.exp2(32.0 * jnp.log2(u)))

    a = jnp.minimum(s1, s2)
    b = jnp.maximum(s1, s2)
    pidx = a * 4.0 - a * (a - 1.0) * 0.5 + (b - a)

    g = [[f1[z] * f2[t] for t in range(4)] for z in range(8)]
    cols = []
    for q in range(_NQ):
        wq = jnp.where(pidx == np.float32(q), base, 0.0)
        for z in range(8):
            for t in range(4):
                cols.append(jnp.sum(wq * g[z][t], axis=1, keepdims=True))
    angular = jnp.concatenate(cols, axis=1)

    out_ref[0] = jnp.concatenate([radial, angular], axis=1)


def kernel(species, positions):
    sp = species[0]
    pos = positions[0]
    n = pos.shape[0]
    posT = pos.T
    spcol = sp[:, None]
    pm = sp[None, :].astype(jnp.float32)
    selj = jnp.asarray(_SELJ6)
    selk = jnp.asarray(_SELK6)

    feats = pl.pallas_call(
        _ani_kernel,
        grid=(n // _BI,),
        in_specs=[
            pl.BlockSpec((_BI, 3), lambda i: (i, 0)),
            pl.BlockSpec((3, n), lambda i: (0, 0)),
            pl.BlockSpec((n, 1), lambda i: (0, 0)),
            pl.BlockSpec((1, n), lambda i: (0, 0)),
            pl.BlockSpec((_NSEG * _K, _NSEG * _PW), lambda i: (0, 0)),
            pl.BlockSpec((_NSEG * _K, _NSEG * _PW), lambda i: (0, 0)),
        ],
        out_specs=pl.BlockSpec((1, _BI, 384), lambda i: (0, i, 0)),
        out_shape=jax.ShapeDtypeStruct((1, n, 384), jnp.float32),
    )(pos, posT, spcol, pm, selj, selk)

    return species, feats


# speedup vs baseline: 1.7670x; 1.0155x over previous
import numpy as np
import jax
import jax.numpy as jnp
from jax.experimental import pallas as pl

_NSP = 4
_RCR = 5.2
_RCA = 3.5
_ETAR = 16.0
_ETAA = 8.0
_SHFR = (0.9 + 0.26875 * np.arange(16)).astype(np.float32)
_SHFA = np.array([0.9, 1.55, 2.2, 2.85], dtype=np.float32)
_SHFZ = ((np.arange(8) + 0.5) * np.pi / 8.0).astype(np.float32)
_COSZ = np.cos(_SHFZ)
_SINZ = np.sin(_SHFZ)
_K = 24
_NQ = 10

_JJ, _KK = np.triu_indices(_K, 1)
_P = _JJ.size
_PW = 384
_NSEG = 6
_SELJ6 = np.zeros((_NSEG * _K, _NSEG * _PW), dtype=np.float32)
_SELK6 = np.zeros((_NSEG * _K, _NSEG * _PW), dtype=np.float32)
for _g in range(_NSEG):
    _SELJ6[_g * _K + _JJ, _g * _PW + np.arange(_P)] = 1.0
    _SELK6[_g * _K + _KK, _g * _PW + np.arange(_P)] = 1.0

_BI = 128


def _cut(d, rc):
    return jnp.where(d < rc, 0.5 * jnp.cos(np.float32(np.pi / rc) * d) + 0.5, 0.0)


def _pow32(x):
    x = x * x
    x = x * x
    x = x * x
    x = x * x
    return x * x


def _ani_kernel(posi_ref, posT_ref, spcol_ref, pm_ref, selj_ref, selk_ref, out_ref):
    i = pl.program_id(0)
    bi = posi_ref.shape[0]
    n = posT_ref.shape[1]

    xi = posi_ref[:, 0:1]
    yi = posi_ref[:, 1:2]
    zi = posi_ref[:, 2:3]
    xr = posT_ref[0:1, :]
    yr = posT_ref[1:2, :]
    zr = posT_ref[2:3, :]

    dxm = xi - xr
    dym = yi - yr
    dzm = zi - zr
    d2m = dxm * dxm + dym * dym + dzm * dzm + 1e-12

    jio_i = jax.lax.broadcasted_iota(jnp.int32, (bi, n), 1)
    row_i = jax.lax.broadcasted_iota(jnp.int32, (bi, n), 0) + i * bi
    d2m = jnp.where(jio_i == row_i, 1e12, d2m)
    dist = jnp.sqrt(d2m)

    fc4 = 0.25 * _cut(dist, _RCR)
    oh = (spcol_ref[:, 0:1] == jax.lax.broadcasted_iota(jnp.int32, (n, _NSP), 1)).astype(jnp.float32)
    rads = []
    for m in range(16):
        dd = dist - _SHFR[m]
        t = jnp.exp(-_ETAR * dd * dd) * fc4
        rads.append(jax.lax.dot_general(t, oh, (((1,), (0,)), ((), ())),
                                        preferred_element_type=jnp.float32))
    radial = jnp.stack(rads, axis=2).reshape(bi, _NSP * 16)

    jio_f = jio_i.astype(jnp.float32)
    dwork = jnp.where(dist < _RCA, dist, 1e6)
    spf_row = pm_ref[0:1, :]
    nds, ndxl, ndyl, ndzl, nspl = [], [], [], [], []
    for _ in range(_K):
        mval = jnp.min(dwork, axis=1, keepdims=True)
        aidx = jnp.min(jnp.where(dwork == mval, jio_f, 1e9), axis=1, keepdims=True)
        sel = (jio_f == aidx)
        nds.append(mval)
        ndxl.append(jnp.sum(jnp.where(sel, dxm, 0.0), axis=1, keepdims=True))
        ndyl.append(jnp.sum(jnp.where(sel, dym, 0.0), axis=1, keepdims=True))
        ndzl.append(jnp.sum(jnp.where(sel, dzm, 0.0), axis=1, keepdims=True))
        nspl.append(jnp.sum(jnp.where(sel, spf_row, 0.0), axis=1, keepdims=True))
        dwork = jnp.where(sel, 1e6, dwork)
    nd = jnp.concatenate(nds, axis=1)
    nx = -jnp.concatenate(ndxl, axis=1)
    ny = -jnp.concatenate(ndyl, axis=1)
    nz = -jnp.concatenate(ndzl, axis=1)
    ns = jnp.concatenate(nspl, axis=1)
    fcn = _cut(nd, _RCA)

    big = jnp.concatenate([nd, nx, ny, nz, ns, fcn], axis=1)
    bj = jax.lax.dot_general(big, selj_ref[...], (((1,), (0,)), ((), ())),
                             preferred_element_type=jnp.float32, precision=jax.lax.Precision.HIGHEST)
    bk = jax.lax.dot_general(big, selk_ref[...], (((1,), (0,)), ((), ())),
                             preferred_element_type=jnp.float32, precision=jax.lax.Precision.HIGHEST)
    d1 = bj[:, 0 * _PW:1 * _PW]
    x1 = bj[:, 1 * _PW:2 * _PW]
    y1 = bj[:, 2 * _PW:3 * _PW]
    z1 = bj[:, 3 * _PW:4 * _PW]
    s1 = bj[:, 4 * _PW:5 * _PW]
    fc1 = bj[:, 5 * _PW:6 * _PW]
    d2 = bk[:, 0 * _PW:1 * _PW]
    x2 = bk[:, 1 * _PW:2 * _PW]
    y2 = bk[:, 2 * _PW:3 * _PW]
    z2 = bk[:, 3 * _PW:4 * _PW]
    s2 = bk[:, 4 * _PW:5 * _PW]
    fc2 = bk[:, 5 * _PW:6 * _PW]

    dotp = x1 * x2 + y1 * y2 + z1 * z2
    c = 0.95 * dotp / (d1 * d2 + 1e-12)
    c = jnp.clip(c, -0.999999, 0.999999)
    s = jnp.sqrt(1.0 - c * c)
    base = 2.0 * fc1 * fc2

    davg = 0.5 * (d1 + d2)
    f2 = []
    for q in range(4):
        dd = davg - _SHFA[q]
        f2.append(jnp.exp(-_ETAA * dd * dd))
    ch = 0.5 * c
    sh = 0.5 * s
    f1 = []
    for z in range(8):
        u = jnp.maximum(0.5 + ch * _COSZ[z] + sh * _SINZ[z], 0.0)
        f1.append(jnp.exp2(32.0 * jnp.log2(u)))

    a = jnp.minimum(s1, s2)
    b = jnp.maximum(s1, s2)
    pidx = a * 4.0 - a * (a - 1.0) * 0.5 + (b - a)

    g = [[f1[z] * f2[t] for t in range(4)] for z in range(8)]
    cols = []
    for q in range(_NQ):
        wq = jnp.where(pidx == np.float32(q), base, 0.0)
        for z in range(8):
            for t in range(4):
                cols.append(jnp.sum(wq * g[z][t], axis=1, keepdims=True))
    angular = jnp.concatenate(cols, axis=1)

    out_ref[0] = jnp.concatenate([radial, angular], axis=1)


def kernel(species, positions):
    sp = species[0]
    pos = positions[0]
    n = pos.shape[0]
    posT = pos.T
    spcol = sp[:, None]
    pm = sp[None, :].astype(jnp.float32)
    selj = jnp.asarray(_SELJ6)
    selk = jnp.asarray(_SELK6)

    feats = pl.pallas_call(
        _ani_kernel,
        grid=(n // _BI,),
        in_specs=[
            pl.BlockSpec((_BI, 3), lambda i: (i, 0)),
            pl.BlockSpec((3, n), lambda i: (0, 0)),
            pl.BlockSpec((n, 1), lambda i: (0, 0)),
            pl.BlockSpec((1, n), lambda i: (0, 0)),
            pl.BlockSpec((_NSEG * _K, _NSEG * _PW), lambda i: (0, 0)),
            pl.BlockSpec((_NSEG * _K, _NSEG * _PW), lambda i: (0, 0)),
        ],
        out_specs=pl.BlockSpec((1, _BI, 384), lambda i: (0, i, 0)),
        out_shape=jax.ShapeDtypeStruct((1, n, 384), jnp.float32),
    )(pos, posT, spcol, pm, selj, selk)

    return species, feats


# speedup vs baseline: 1.7679x; 1.0005x over previous
import numpy as np
import jax
import jax.numpy as jnp
from jax.experimental import pallas as pl

_NSP = 4
_RCR = 5.2
_RCA = 3.5
_ETAR = 16.0
_ETAA = 8.0
_SHFR = (0.9 + 0.26875 * np.arange(16)).astype(np.float32)
_SHFA = np.array([0.9, 1.55, 2.2, 2.85], dtype=np.float32)
_SHFZ = ((np.arange(8) + 0.5) * np.pi / 8.0).astype(np.float32)
_COSZ = np.cos(_SHFZ)
_SINZ = np.sin(_SHFZ)
_K = 24
_NQ = 10

_JJ, _KK = np.triu_indices(_K, 1)
_P = _JJ.size
_PW = 384
_NSEG = 6
_SELJ6 = np.zeros((_NSEG * _K, _NSEG * _PW), dtype=np.float32)
_SELK6 = np.zeros((_NSEG * _K, _NSEG * _PW), dtype=np.float32)
for _g in range(_NSEG):
    _SELJ6[_g * _K + _JJ, _g * _PW + np.arange(_P)] = 1.0
    _SELK6[_g * _K + _KK, _g * _PW + np.arange(_P)] = 1.0

_BI = 128


def _cut(d, rc):
    return jnp.where(d < rc, 0.5 * jnp.cos(np.float32(np.pi / rc) * d) + 0.5, 0.0)


def _ani_kernel(posi_ref, posT_ref, spcol_ref, pm_ref, selj_ref, selk_ref, out_ref):
    i = pl.program_id(0)
    bi = posi_ref.shape[0]
    n = posT_ref.shape[1]

    xi = posi_ref[:, 0:1]
    yi = posi_ref[:, 1:2]
    zi = posi_ref[:, 2:3]
    xr = posT_ref[0:1, :]
    yr = posT_ref[1:2, :]
    zr = posT_ref[2:3, :]

    dxm = xi - xr
    dym = yi - yr
    dzm = zi - zr
    d2m = dxm * dxm + dym * dym + dzm * dzm + 1e-12

    jio_i = jax.lax.broadcasted_iota(jnp.int32, (bi, n), 1)
    row_i = jax.lax.broadcasted_iota(jnp.int32, (bi, n), 0) + i * bi
    d2m = jnp.where(jio_i == row_i, 1e12, d2m)
    dist = jnp.sqrt(d2m)

    fc4 = 0.25 * _cut(dist, _RCR)
    oh = (spcol_ref[:, 0:1] == jax.lax.broadcasted_iota(jnp.int32, (n, _NSP), 1)).astype(jnp.float32)
    rads = []
    for m in range(16):
        dd = dist - _SHFR[m]
        t = jnp.exp(-_ETAR * dd * dd) * fc4
        rads.append(jax.lax.dot_general(t, oh, (((1,), (0,)), ((), ())),
                                        preferred_element_type=jnp.float32))
    radial = jnp.stack(rads, axis=2).reshape(bi, _NSP * 16)

    jio_f = jio_i.astype(jnp.float32)
    dwork = jnp.where(dist < _RCA, dist, 1e6)
    spf_row = pm_ref[0:1, :]
    nds, ndxl, ndyl, ndzl, nspl = [], [], [], [], []
    for _ in range(_K):
        mval = jnp.min(dwork, axis=1, keepdims=True)
        aidx = jnp.min(jnp.where(dwork == mval, jio_f, 1e9), axis=1, keepdims=True)
        sel = (jio_f == aidx)
        nds.append(mval)
        ndxl.append(jnp.sum(jnp.where(sel, dxm, 0.0), axis=1, keepdims=True))
        ndyl.append(jnp.sum(jnp.where(sel, dym, 0.0), axis=1, keepdims=True))
        ndzl.append(jnp.sum(jnp.where(sel, dzm, 0.0), axis=1, keepdims=True))
        nspl.append(jnp.sum(jnp.where(sel, spf_row, 0.0), axis=1, keepdims=True))
        dwork = jnp.where(sel, 1e6, dwork)
    nd = jnp.concatenate(nds, axis=1)
    nx = -jnp.concatenate(ndxl, axis=1)
    ny = -jnp.concatenate(ndyl, axis=1)
    nz = -jnp.concatenate(ndzl, axis=1)
    ns = jnp.concatenate(nspl, axis=1)
    fcn = _cut(nd, _RCA)

    big = jnp.concatenate([nd, nx, ny, nz, ns, fcn], axis=1)
    bj = jax.lax.dot_general(big, selj_ref[...], (((1,), (0,)), ((), ())),
                             preferred_element_type=jnp.float32, precision=jax.lax.Precision.HIGHEST)
    bk = jax.lax.dot_general(big, selk_ref[...], (((1,), (0,)), ((), ())),
                             preferred_element_type=jnp.float32, precision=jax.lax.Precision.HIGHEST)
    d1 = bj[:, 0 * _PW:1 * _PW]
    x1 = bj[:, 1 * _PW:2 * _PW]
    y1 = bj[:, 2 * _PW:3 * _PW]
    z1 = bj[:, 3 * _PW:4 * _PW]
    s1 = bj[:, 4 * _PW:5 * _PW]
    fc1 = bj[:, 5 * _PW:6 * _PW]
    d2 = bk[:, 0 * _PW:1 * _PW]
    x2 = bk[:, 1 * _PW:2 * _PW]
    y2 = bk[:, 2 * _PW:3 * _PW]
    z2 = bk[:, 3 * _PW:4 * _PW]
    s2 = bk[:, 4 * _PW:5 * _PW]
    fc2 = bk[:, 5 * _PW:6 * _PW]

    dotp = x1 * x2 + y1 * y2 + z1 * z2
    c = 0.95 * dotp / (d1 * d2 + 1e-12)
    c = jnp.clip(c, -0.999999, 0.999999)
    s = jnp.sqrt(1.0 - c * c)
    base = 2.0 * fc1 * fc2

    davg = 0.5 * (d1 + d2)
    f2 = []
    for q in range(4):
        dd = davg - _SHFA[q]
        f2.append(jnp.exp(-_ETAA * dd * dd))
    ch = 0.5 * c
    sh = 0.5 * s
    f1 = []
    for z in range(8):
        u = jnp.maximum(0.5 + ch * _COSZ[z] + sh * _SINZ[z], 0.0)
        f1.append(jnp.exp2(32.0 * jnp.log2(u)))

    a = jnp.minimum(s1, s2)
    b = jnp.maximum(s1, s2)
    pidx = a * 4.0 - a * (a - 1.0) * 0.5 + (b - a)

    g = [[f1[z] * f2[t] for t in range(4)] for z in range(8)]
    cols = []
    for q in range(_NQ):
        wq = jnp.where(pidx == np.float32(q), base, 0.0)
        for z in range(8):
            for t in range(4):
                cols.append(jnp.sum(wq * g[z][t], axis=1, keepdims=True))
    angular = jnp.concatenate(cols, axis=1)

    out_ref[0] = jnp.concatenate([radial, angular], axis=1)


def kernel(species, positions):
    sp = species[0]
    pos = positions[0]
    n = pos.shape[0]
    posT = pos.T
    spcol = sp[:, None]
    pm = sp[None, :].astype(jnp.float32)
    selj = jnp.asarray(_SELJ6)
    selk = jnp.asarray(_SELK6)

    feats = pl.pallas_call(
        _ani_kernel,
        grid=(n // _BI,),
        in_specs=[
            pl.BlockSpec((_BI, 3), lambda i: (i, 0)),
            pl.BlockSpec((3, n), lambda i: (0, 0)),
            pl.BlockSpec((n, 1), lambda i: (0, 0)),
            pl.BlockSpec((1, n), lambda i: (0, 0)),
            pl.BlockSpec((_NSEG * _K, _NSEG * _PW), lambda i: (0, 0)),
            pl.BlockSpec((_NSEG * _K, _NSEG * _PW), lambda i: (0, 0)),
        ],
        out_specs=pl.BlockSpec((1, _BI, 384), lambda i: (0, i, 0)),
        out_shape=jax.ShapeDtypeStruct((1, n, 384), jnp.float32),
    )(pos, posT, spcol, pm, selj, selk)

    return species, feats
